# baseline (speedup 1.0000x reference)
import functools

import jax
import jax.numpy as jnp
from jax import lax
from jax.experimental import pallas as pl
from jax.experimental.pallas import tpu as pltpu
from jax.experimental.pallas import tpu_sc as plsc

N_NODES = 10000
N_EDGES = 320000
D = 128

NUM_CORES = 2
NUM_SUBCORES = 16
E_PER_TILE = N_EDGES // NUM_SUBCORES
ROWS_PER_TILE = N_NODES // NUM_SUBCORES

W = 64
DEGW = 16
CHUNK = 200
N_CHUNKS = E_PER_TILE // CHUNK


def _make_sc_aggregate(with_deg: bool):
  mesh = plsc.VectorSubcoreMesh(core_axis_name="c", subcore_axis_name="s",
                                num_cores=NUM_CORES, num_subcores=NUM_SUBCORES)

  out_type = [jax.ShapeDtypeStruct((N_NODES, D), jnp.float32)]
  scratch = [
      pltpu.VMEM((CHUNK,), jnp.int32),
      pltpu.VMEM((CHUNK,), jnp.int32),
      pltpu.VMEM((CHUNK,), jnp.int32),
      pltpu.VMEM((CHUNK,), jnp.int32),
      pltpu.VMEM((CHUNK, W), jnp.float32),
      pltpu.VMEM((CHUNK, W), jnp.float32),
      pltpu.VMEM_SHARED((N_NODES, W), jnp.float32),
      pltpu.SemaphoreType.DMA,
      pltpu.SemaphoreType.DMA,
      pltpu.SemaphoreType.DMA,
      pltpu.SemaphoreType.DMA,
  ]
  if with_deg:
    out_type.append(jax.ShapeDtypeStruct((N_NODES, D), jnp.float32))
    scratch += [
        pltpu.VMEM((CHUNK, DEGW), jnp.float32),
        pltpu.VMEM_SHARED((N_NODES, DEGW), jnp.float32),
    ]

  @functools.partial(
      pl.kernel,
      out_type=tuple(out_type),
      mesh=mesh,
      scratch_types=scratch,
      compiler_params=pltpu.CompilerParams(use_tc_tiling_on_sc=False),
  )
  def sc_agg(t0_hbm, t1_hbm, eidx_hbm, zeros_hbm, ones_hbm, *rest):
    if with_deg:
      out_hbm, deg_hbm = rest[0], rest[1]
      (src0, src1, dst0, dst1, buf0, buf1, acc_sh,
       isem0, isem1, rsem0, rsem1, ones_v, deg_sh) = rest[2:]
    else:
      out_hbm = rest[0]
      (src0, src1, dst0, dst1, buf0, buf1, acc_sh,
       isem0, isem1, rsem0, rsem1) = rest[1:]

    cid = lax.axis_index("c")
    sid = lax.axis_index("s")
    row0 = sid * ROWS_PER_TILE
    base = sid * E_PER_TILE
    rows = pl.ds(row0, ROWS_PER_TILE)

    ibufs = ((src0, dst0), (src1, dst1))
    isems = (isem0, isem1)
    rbufs = (buf0, buf1)
    rsems = (rsem0, rsem1)

    pltpu.sync_copy(zeros_hbm.at[rows], acc_sh.at[rows])
    if with_deg:
      pltpu.sync_copy(zeros_hbm.at[rows, pl.ds(0, DEGW)], deg_sh.at[rows])
      pltpu.sync_copy(ones_hbm, ones_v)
    plsc.subcore_barrier()

    def idx_load(k, b):
      off = base + k * CHUNK
      return (pltpu.make_async_copy(eidx_hbm.at[0, pl.ds(off, CHUNK)],
                                    ibufs[b][0], isems[b]),
              pltpu.make_async_copy(eidx_hbm.at[1, pl.ds(off, CHUNK)],
                                    ibufs[b][1], isems[b]))

    def run_edges(table_hbm, half):
      def gather(b):
        return pltpu.make_async_copy(table_hbm.at[ibufs[b][0]], rbufs[b],
                                     rsems[b])

      ia, ib = idx_load(0, 0)
      ia.start(); ib.start(); ia.wait(); ib.wait()
      ia, ib = idx_load(1, 1)
      ia.start(); ib.start()
      gather(0).start()

      def body(j, carry):
        for b in range(2):
          k = 2 * j + b

          @pl.when(k + 1 < N_CHUNKS)
          def _():
            ia, ib = idx_load(k + 1, 1 - b)
            ia.wait(); ib.wait()
            gather(1 - b).start()

          gather(b).wait()
          pltpu.sync_copy(rbufs[b], acc_sh.at[ibufs[b][1]], add=True)
          if with_deg and b == half:
            pltpu.sync_copy(ones_v, deg_sh.at[ibufs[b][1]], add=True)

          @pl.when(k + 2 < N_CHUNKS)
          def _():
            ia, ib = idx_load(k + 2, b)
            ia.start(); ib.start()
        return carry

      lax.fori_loop(0, N_CHUNKS // 2, body, 0, unroll=False)

    @pl.when(cid == 0)
    def _():
      run_edges(t0_hbm, 0)

    @pl.when(cid == 1)
    def _():
      run_edges(t1_hbm, 1)

    plsc.subcore_barrier()

    @pl.when(cid == 0)
    def _():
      pltpu.sync_copy(acc_sh.at[rows], out_hbm.at[rows, pl.ds(0, W)])
      if with_deg:
        pltpu.sync_copy(deg_sh.at[rows], deg_hbm.at[rows, pl.ds(0, DEGW)])

    @pl.when(cid == 1)
    def _():
      pltpu.sync_copy(acc_sh.at[rows], out_hbm.at[rows, pl.ds(W, W)])
      if with_deg:
        pltpu.sync_copy(deg_sh.at[rows], deg_hbm.at[rows, pl.ds(DEGW, DEGW)])

  return sc_agg


_sc_agg_cache = {}


def _sc_aggregate(with_deg: bool):
  if with_deg not in _sc_agg_cache:
    _sc_agg_cache[with_deg] = _make_sc_aggregate(with_deg)
  return _sc_agg_cache[with_deg]


ROW_BLK = 2000


def _tc_layer1_body(agg_ref, deg_ref, x_ref, ws_ref, wn_ref, b_ref,
                    h1a_ref, h1b_ref, deginv_ref):
  deg = deg_ref[:, 0:1] + deg_ref[:, DEGW:DEGW + 1]
  scale = 1.0 / jnp.maximum(deg, 1.0)
  hn = agg_ref[...] * scale
  acc = jnp.dot(x_ref[...], ws_ref[...], preferred_element_type=jnp.float32)
  acc += jnp.dot(hn, wn_ref[...], preferred_element_type=jnp.float32)
  acc += b_ref[...]
  h1 = jnp.maximum(acc, 0.0)
  h1a_ref[...] = h1[:, 0:64]
  h1b_ref[...] = h1[:, 64:128]
  deginv_ref[...] = scale


def _tc_layer2_body(agg_ref, h1a_ref, h1b_ref, deginv_ref, ws_ref, wn_ref,
                    b_ref, out_ref):
  h1 = jnp.concatenate([h1a_ref[...], h1b_ref[...]], axis=1)
  hn = agg_ref[...] * deginv_ref[...]
  acc = jnp.dot(h1, ws_ref[...], preferred_element_type=jnp.float32)
  acc += jnp.dot(hn, wn_ref[...], preferred_element_type=jnp.float32)
  acc += b_ref[...]
  out_ref[...] = acc


def _row_spec(width):
  return pl.BlockSpec((ROW_BLK, width), lambda i: (i, 0))


def _full_spec(shape):
  return pl.BlockSpec(shape, lambda i: (0,) * len(shape))


def _tc_layer1(agg, degblk, x, ws, wn, b):
  return pl.pallas_call(
      _tc_layer1_body,
      grid=(N_NODES // ROW_BLK,),
      in_specs=[
          _row_spec(D), _row_spec(D), _row_spec(D),
          _full_spec((D, D)), _full_spec((D, D)), _full_spec((1, D)),
      ],
      out_specs=[_row_spec(64), _row_spec(64), _row_spec(1)],
      out_shape=[
          jax.ShapeDtypeStruct((N_NODES, 64), jnp.float32),
          jax.ShapeDtypeStruct((N_NODES, 64), jnp.float32),
          jax.ShapeDtypeStruct((N_NODES, 1), jnp.float32),
      ],
  )(agg, degblk, x, ws, wn, b)


def _tc_layer2(agg, h1a, h1b, deginv, ws, wn, b):
  return pl.pallas_call(
      _tc_layer2_body,
      grid=(N_NODES // ROW_BLK,),
      in_specs=[
          _row_spec(D), _row_spec(64), _row_spec(64), _row_spec(1),
          _full_spec((D, D)), _full_spec((D, D)), _full_spec((1, D)),
      ],
      out_specs=_row_spec(D),
      out_shape=jax.ShapeDtypeStruct((N_NODES, D), jnp.float32),
  )(agg, h1a, h1b, deginv, ws, wn, b)


@jax.jit
def kernel(inputs, edge_index, W_self1, W_neigh1, b1, W_self2, W_neigh2, b2):
  t0 = inputs[:, 0:W]
  t1 = inputs[:, W:D]

  zeros = jnp.zeros((N_NODES, W), jnp.float32)
  ones_rows = jnp.ones((CHUNK, DEGW), jnp.float32)

  agg1, degblk = _sc_aggregate(True)(t0, t1, edge_index, zeros, ones_rows)
  h1a, h1b, deginv = _tc_layer1(agg1, degblk, inputs,
                                W_self1, W_neigh1, b1.reshape(1, D))

  agg2, = _sc_aggregate(False)(h1a, h1b, edge_index, zeros, ones_rows)
  out = _tc_layer2(agg2, h1a, h1b, deginv,
                   W_self2, W_neigh2, b2.reshape(1, D))
  return out


# speedup vs baseline: 1.1560x; 1.1560x over previous
import functools

import jax
import jax.numpy as jnp
from jax import lax
from jax.experimental import pallas as pl
from jax.experimental.pallas import tpu as pltpu
from jax.experimental.pallas import tpu_sc as plsc

N_NODES = 10000
N_EDGES = 320000
D = 128

NUM_CORES = 2
NUM_SUBCORES = 16
E_PER_TILE = N_EDGES // NUM_SUBCORES
ROWS_PER_TILE = N_NODES // NUM_SUBCORES

W = 64
DEGW = 8
CHUNK = 400
N_CHUNKS = E_PER_TILE // CHUNK


def _make_sc_aggregate(with_deg: bool):
  mesh = plsc.VectorSubcoreMesh(core_axis_name="c", subcore_axis_name="s",
                                num_cores=NUM_CORES, num_subcores=NUM_SUBCORES)

  out_type = [jax.ShapeDtypeStruct((N_NODES, D), jnp.float32)]
  scratch = [
      pltpu.VMEM((CHUNK,), jnp.int32),
      pltpu.VMEM((CHUNK,), jnp.int32),
      pltpu.VMEM((CHUNK,), jnp.int32),
      pltpu.VMEM((CHUNK,), jnp.int32),
      pltpu.VMEM((CHUNK, W), jnp.float32),
      pltpu.VMEM((CHUNK, W), jnp.float32),
      pltpu.VMEM_SHARED((N_NODES, W), jnp.float32),
      pltpu.SemaphoreType.DMA,
      pltpu.SemaphoreType.DMA,
      pltpu.SemaphoreType.DMA,
      pltpu.SemaphoreType.DMA,
  ]
  if with_deg:
    out_type.append(jax.ShapeDtypeStruct((N_NODES, D), jnp.float32))
    scratch += [
        pltpu.VMEM((CHUNK, DEGW), jnp.float32),
        pltpu.VMEM_SHARED((N_NODES, DEGW), jnp.float32),
    ]

  @functools.partial(
      pl.kernel,
      out_type=tuple(out_type),
      mesh=mesh,
      scratch_types=scratch,
      compiler_params=pltpu.CompilerParams(use_tc_tiling_on_sc=False),
  )
  def sc_agg(t0_hbm, t1_hbm, eidx_hbm, zeros_hbm, ones_hbm, *rest):
    if with_deg:
      out_hbm, deg_hbm = rest[0], rest[1]
      (src0, src1, dst0, dst1, buf0, buf1, acc_sh,
       isem0, isem1, rsem0, rsem1, ones_v, deg_sh) = rest[2:]
    else:
      out_hbm = rest[0]
      (src0, src1, dst0, dst1, buf0, buf1, acc_sh,
       isem0, isem1, rsem0, rsem1) = rest[1:]

    cid = lax.axis_index("c")
    sid = lax.axis_index("s")
    row0 = sid * ROWS_PER_TILE
    base = sid * E_PER_TILE
    rows = pl.ds(row0, ROWS_PER_TILE)

    ibufs = ((src0, dst0), (src1, dst1))
    isems = (isem0, isem1)
    rbufs = (buf0, buf1)
    rsems = (rsem0, rsem1)

    pltpu.sync_copy(zeros_hbm.at[rows], acc_sh.at[rows])
    if with_deg:
      pltpu.sync_copy(zeros_hbm.at[rows, pl.ds(0, DEGW)], deg_sh.at[rows])
      pltpu.sync_copy(ones_hbm, ones_v)
    plsc.subcore_barrier()

    def idx_load(k, b):
      off = base + k * CHUNK
      return (pltpu.make_async_copy(eidx_hbm.at[0, pl.ds(off, CHUNK)],
                                    ibufs[b][0], isems[b]),
              pltpu.make_async_copy(eidx_hbm.at[1, pl.ds(off, CHUNK)],
                                    ibufs[b][1], isems[b]))

    def run_edges(table_hbm, half):
      def gather(b):
        return pltpu.make_async_copy(table_hbm.at[ibufs[b][0]], rbufs[b],
                                     rsems[b])

      ia, ib = idx_load(0, 0)
      ia.start(); ib.start(); ia.wait(); ib.wait()
      ia, ib = idx_load(1, 1)
      ia.start(); ib.start()
      gather(0).start()

      def body(j, carry):
        for b in range(2):
          k = 2 * j + b

          @pl.when(k + 1 < N_CHUNKS)
          def _():
            ia, ib = idx_load(k + 1, 1 - b)
            ia.wait(); ib.wait()
            gather(1 - b).start()

          gather(b).wait()
          pltpu.sync_copy(rbufs[b], acc_sh.at[ibufs[b][1]], add=True)
          if with_deg and b == half:
            pltpu.sync_copy(ones_v, deg_sh.at[ibufs[b][1]], add=True)

          @pl.when(k + 2 < N_CHUNKS)
          def _():
            ia, ib = idx_load(k + 2, b)
            ia.start(); ib.start()
        return carry

      lax.fori_loop(0, N_CHUNKS // 2, body, 0, unroll=False)

    @pl.when(cid == 0)
    def _():
      run_edges(t0_hbm, 0)

    @pl.when(cid == 1)
    def _():
      run_edges(t1_hbm, 1)

    plsc.subcore_barrier()

    @pl.when(cid == 0)
    def _():
      pltpu.sync_copy(acc_sh.at[rows], out_hbm.at[rows, pl.ds(0, W)])
      if with_deg:
        pltpu.sync_copy(deg_sh.at[rows], deg_hbm.at[rows, pl.ds(0, DEGW)])

    @pl.when(cid == 1)
    def _():
      pltpu.sync_copy(acc_sh.at[rows], out_hbm.at[rows, pl.ds(W, W)])
      if with_deg:
        pltpu.sync_copy(deg_sh.at[rows], deg_hbm.at[rows, pl.ds(DEGW, DEGW)])

  return sc_agg


_sc_agg_cache = {}


def _sc_aggregate(with_deg: bool):
  if with_deg not in _sc_agg_cache:
    _sc_agg_cache[with_deg] = _make_sc_aggregate(with_deg)
  return _sc_agg_cache[with_deg]


ROW_BLK = 2000


def _tc_layer1_body(agg_ref, deg_ref, x_ref, ws_ref, wn_ref, b_ref,
                    h1a_ref, h1b_ref, deginv_ref):
  deg = deg_ref[:, 0:1] + deg_ref[:, DEGW:DEGW + 1]
  scale = 1.0 / jnp.maximum(deg, 1.0)
  hn = agg_ref[...] * scale
  acc = jnp.dot(x_ref[...], ws_ref[...], preferred_element_type=jnp.float32)
  acc += jnp.dot(hn, wn_ref[...], preferred_element_type=jnp.float32)
  acc += b_ref[...]
  h1 = jnp.maximum(acc, 0.0)
  h1a_ref[...] = h1[:, 0:64]
  h1b_ref[...] = h1[:, 64:128]
  deginv_ref[...] = scale


def _tc_layer2_body(agg_ref, h1a_ref, h1b_ref, deginv_ref, ws_ref, wn_ref,
                    b_ref, out_ref):
  h1 = jnp.concatenate([h1a_ref[...], h1b_ref[...]], axis=1)
  hn = agg_ref[...] * deginv_ref[...]
  acc = jnp.dot(h1, ws_ref[...], preferred_element_type=jnp.float32)
  acc += jnp.dot(hn, wn_ref[...], preferred_element_type=jnp.float32)
  acc += b_ref[...]
  out_ref[...] = acc


def _row_spec(width):
  return pl.BlockSpec((ROW_BLK, width), lambda i: (i, 0))


def _full_spec(shape):
  return pl.BlockSpec(shape, lambda i: (0,) * len(shape))


def _tc_layer1(agg, degblk, x, ws, wn, b):
  return pl.pallas_call(
      _tc_layer1_body,
      grid=(N_NODES // ROW_BLK,),
      in_specs=[
          _row_spec(D), _row_spec(D), _row_spec(D),
          _full_spec((D, D)), _full_spec((D, D)), _full_spec((1, D)),
      ],
      out_specs=[_row_spec(64), _row_spec(64), _row_spec(1)],
      out_shape=[
          jax.ShapeDtypeStruct((N_NODES, 64), jnp.float32),
          jax.ShapeDtypeStruct((N_NODES, 64), jnp.float32),
          jax.ShapeDtypeStruct((N_NODES, 1), jnp.float32),
      ],
  )(agg, degblk, x, ws, wn, b)


def _tc_layer2(agg, h1a, h1b, deginv, ws, wn, b):
  return pl.pallas_call(
      _tc_layer2_body,
      grid=(N_NODES // ROW_BLK,),
      in_specs=[
          _row_spec(D), _row_spec(64), _row_spec(64), _row_spec(1),
          _full_spec((D, D)), _full_spec((D, D)), _full_spec((1, D)),
      ],
      out_specs=_row_spec(D),
      out_shape=jax.ShapeDtypeStruct((N_NODES, D), jnp.float32),
  )(agg, h1a, h1b, deginv, ws, wn, b)


@jax.jit
def kernel(inputs, edge_index, W_self1, W_neigh1, b1, W_self2, W_neigh2, b2):
  t0 = inputs[:, 0:W]
  t1 = inputs[:, W:D]

  zeros = jnp.zeros((N_NODES, W), jnp.float32)
  ones_rows = jnp.ones((CHUNK, DEGW), jnp.float32)

  agg1, degblk = _sc_aggregate(True)(t0, t1, edge_index, zeros, ones_rows)
  h1a, h1b, deginv = _tc_layer1(agg1, degblk, inputs,
                                W_self1, W_neigh1, b1.reshape(1, D))

  agg2, = _sc_aggregate(False)(h1a, h1b, edge_index, zeros, ones_rows)
  out = _tc_layer2(agg2, h1a, h1b, deginv,
                   W_self2, W_neigh2, b2.reshape(1, D))
  return out


# speedup vs baseline: 1.1834x; 1.0237x over previous
import functools

import jax
import jax.numpy as jnp
from jax import lax
from jax.experimental import pallas as pl
from jax.experimental.pallas import tpu as pltpu
from jax.experimental.pallas import tpu_sc as plsc

N_NODES = 10000
N_EDGES = 320000
D = 128

NUM_CORES = 2
NUM_SUBCORES = 16
E_PER_TILE = N_EDGES // NUM_SUBCORES
ROWS_PER_TILE = N_NODES // NUM_SUBCORES

W = 64
DEGW = 8
CHUNK = 400
N_CHUNKS = E_PER_TILE // CHUNK


def _make_sc_aggregate(with_deg: bool):
  mesh = plsc.VectorSubcoreMesh(core_axis_name="c", subcore_axis_name="s",
                                num_cores=NUM_CORES, num_subcores=NUM_SUBCORES)

  out_type = [jax.ShapeDtypeStruct((N_NODES, D), jnp.float32)]
  scratch = [
      pltpu.VMEM((CHUNK,), jnp.int32),
      pltpu.VMEM((CHUNK,), jnp.int32),
      pltpu.VMEM((CHUNK,), jnp.int32),
      pltpu.VMEM((CHUNK,), jnp.int32),
      pltpu.VMEM((CHUNK, W), jnp.float32),
      pltpu.VMEM((CHUNK, W), jnp.float32),
      pltpu.VMEM_SHARED((N_NODES, W), jnp.float32),
      pltpu.SemaphoreType.DMA,
      pltpu.SemaphoreType.DMA,
      pltpu.SemaphoreType.DMA,
      pltpu.SemaphoreType.DMA,
      pltpu.SemaphoreType.DMA,
  ]
  if with_deg:
    out_type.append(jax.ShapeDtypeStruct((N_NODES, D), jnp.float32))
    scratch += [
        pltpu.VMEM((CHUNK, DEGW), jnp.float32),
        pltpu.VMEM_SHARED((N_NODES, DEGW), jnp.float32),
    ]

  @functools.partial(
      pl.kernel,
      out_type=tuple(out_type),
      mesh=mesh,
      scratch_types=scratch,
      compiler_params=pltpu.CompilerParams(use_tc_tiling_on_sc=False),
  )
  def sc_agg(t0_hbm, t1_hbm, eidx_hbm, zeros_hbm, ones_hbm, *rest):
    if with_deg:
      out_hbm, deg_hbm = rest[0], rest[1]
      (src0, src1, dst0, dst1, buf0, buf1, acc_sh,
       isem0, isem1, rsem0, rsem1, zsem, ones_v, deg_sh) = rest[2:]
    else:
      out_hbm = rest[0]
      (src0, src1, dst0, dst1, buf0, buf1, acc_sh,
       isem0, isem1, rsem0, rsem1, zsem) = rest[1:]

    cid = lax.axis_index("c")
    sid = lax.axis_index("s")
    row0 = sid * ROWS_PER_TILE
    base = sid * E_PER_TILE
    rows = pl.ds(row0, ROWS_PER_TILE)

    ibufs = ((src0, dst0), (src1, dst1))
    isems = (isem0, isem1)
    rbufs = (buf0, buf1)
    rsems = (rsem0, rsem1)

    zcopies = [pltpu.make_async_copy(zeros_hbm.at[rows], acc_sh.at[rows],
                                     zsem)]
    if with_deg:
      zcopies.append(pltpu.make_async_copy(
          zeros_hbm.at[rows, pl.ds(0, DEGW)], deg_sh.at[rows], zsem))
      pltpu.sync_copy(ones_hbm, ones_v)
    for z in zcopies:
      z.start()

    def idx_load(k, b):
      off = base + k * CHUNK
      return (pltpu.make_async_copy(eidx_hbm.at[0, pl.ds(off, CHUNK)],
                                    ibufs[b][0], isems[b]),
              pltpu.make_async_copy(eidx_hbm.at[1, pl.ds(off, CHUNK)],
                                    ibufs[b][1], isems[b]))

    def run_edges(table_hbm, half):
      def gather(b):
        return pltpu.make_async_copy(table_hbm.at[ibufs[b][0]], rbufs[b],
                                     rsems[b])

      ia, ib = idx_load(0, 0)
      ia.start(); ib.start(); ia.wait(); ib.wait()
      ia, ib = idx_load(1, 1)
      ia.start(); ib.start()
      gather(0).start()
      for z in zcopies:
        z.wait()
      plsc.subcore_barrier()

      def body(j, carry):
        for b in range(2):
          k = 2 * j + b

          @pl.when(k + 1 < N_CHUNKS)
          def _():
            ia, ib = idx_load(k + 1, 1 - b)
            ia.wait(); ib.wait()
            gather(1 - b).start()

          gather(b).wait()
          pltpu.sync_copy(rbufs[b], acc_sh.at[ibufs[b][1]], add=True)
          if with_deg and b == half:
            pltpu.sync_copy(ones_v, deg_sh.at[ibufs[b][1]], add=True)

          @pl.when(k + 2 < N_CHUNKS)
          def _():
            ia, ib = idx_load(k + 2, b)
            ia.start(); ib.start()
        return carry

      lax.fori_loop(0, N_CHUNKS // 2, body, 0, unroll=False)

    @pl.when(cid == 0)
    def _():
      run_edges(t0_hbm, 0)

    @pl.when(cid == 1)
    def _():
      run_edges(t1_hbm, 1)

    plsc.subcore_barrier()

    @pl.when(cid == 0)
    def _():
      pltpu.sync_copy(acc_sh.at[rows], out_hbm.at[rows, pl.ds(0, W)])
      if with_deg:
        pltpu.sync_copy(deg_sh.at[rows], deg_hbm.at[rows, pl.ds(0, DEGW)])

    @pl.when(cid == 1)
    def _():
      pltpu.sync_copy(acc_sh.at[rows], out_hbm.at[rows, pl.ds(W, W)])
      if with_deg:
        pltpu.sync_copy(deg_sh.at[rows], deg_hbm.at[rows, pl.ds(DEGW, DEGW)])

  return sc_agg


_sc_agg_cache = {}


def _sc_aggregate(with_deg: bool):
  if with_deg not in _sc_agg_cache:
    _sc_agg_cache[with_deg] = _make_sc_aggregate(with_deg)
  return _sc_agg_cache[with_deg]


ROW_BLK = 5000


def _tc_layer1_body(agg_ref, deg_ref, x_ref, ws_ref, wn_ref, b_ref,
                    h1a_ref, h1b_ref, deginv_ref):
  deg = deg_ref[:, 0:1] + deg_ref[:, DEGW:DEGW + 1]
  scale = 1.0 / jnp.maximum(deg, 1.0)
  hn = agg_ref[...] * scale
  acc = jnp.dot(x_ref[...], ws_ref[...], preferred_element_type=jnp.float32)
  acc += jnp.dot(hn, wn_ref[...], preferred_element_type=jnp.float32)
  acc += b_ref[...]
  h1 = jnp.maximum(acc, 0.0)
  h1a_ref[...] = h1[:, 0:64]
  h1b_ref[...] = h1[:, 64:128]
  deginv_ref[...] = scale


def _tc_layer2_body(agg_ref, h1a_ref, h1b_ref, deginv_ref, ws_ref, wn_ref,
                    b_ref, out_ref):
  h1 = jnp.concatenate([h1a_ref[...], h1b_ref[...]], axis=1)
  hn = agg_ref[...] * deginv_ref[...]
  acc = jnp.dot(h1, ws_ref[...], preferred_element_type=jnp.float32)
  acc += jnp.dot(hn, wn_ref[...], preferred_element_type=jnp.float32)
  acc += b_ref[...]
  out_ref[...] = acc


def _row_spec(width):
  return pl.BlockSpec((ROW_BLK, width), lambda i: (i, 0))


def _full_spec(shape):
  return pl.BlockSpec(shape, lambda i: (0,) * len(shape))


def _tc_layer1(agg, degblk, x, ws, wn, b):
  return pl.pallas_call(
      _tc_layer1_body,
      grid=(N_NODES // ROW_BLK,),
      in_specs=[
          _row_spec(D), _row_spec(D), _row_spec(D),
          _full_spec((D, D)), _full_spec((D, D)), _full_spec((1, D)),
      ],
      out_specs=[_row_spec(64), _row_spec(64), _row_spec(1)],
      out_shape=[
          jax.ShapeDtypeStruct((N_NODES, 64), jnp.float32),
          jax.ShapeDtypeStruct((N_NODES, 64), jnp.float32),
          jax.ShapeDtypeStruct((N_NODES, 1), jnp.float32),
      ],
  )(agg, degblk, x, ws, wn, b)


def _tc_layer2(agg, h1a, h1b, deginv, ws, wn, b):
  return pl.pallas_call(
      _tc_layer2_body,
      grid=(N_NODES // ROW_BLK,),
      in_specs=[
          _row_spec(D), _row_spec(64), _row_spec(64), _row_spec(1),
          _full_spec((D, D)), _full_spec((D, D)), _full_spec((1, D)),
      ],
      out_specs=_row_spec(D),
      out_shape=jax.ShapeDtypeStruct((N_NODES, D), jnp.float32),
  )(agg, h1a, h1b, deginv, ws, wn, b)


@jax.jit
def kernel(inputs, edge_index, W_self1, W_neigh1, b1, W_self2, W_neigh2, b2):
  t0 = inputs[:, 0:W]
  t1 = inputs[:, W:D]

  zeros = jnp.zeros((N_NODES, W), jnp.float32)
  ones_rows = jnp.ones((CHUNK, DEGW), jnp.float32)

  agg1, degblk = _sc_aggregate(True)(t0, t1, edge_index, zeros, ones_rows)
  h1a, h1b, deginv = _tc_layer1(agg1, degblk, inputs,
                                W_self1, W_neigh1, b1.reshape(1, D))

  agg2, = _sc_aggregate(False)(h1a, h1b, edge_index, zeros, ones_rows)
  out = _tc_layer2(agg2, h1a, h1b, deginv,
                   W_self2, W_neigh2, b2.reshape(1, D))
  return out


# speedup vs baseline: 1.2710x; 1.0740x over previous
import functools

import jax
import jax.numpy as jnp
from jax import lax
from jax.experimental import pallas as pl
from jax.experimental.pallas import tpu as pltpu
from jax.experimental.pallas import tpu_sc as plsc

N_NODES = 10000
N_EDGES = 320000
D = 128

NUM_CORES = 2
NUM_SUBCORES = 16
E_PER_TILE = N_EDGES // NUM_SUBCORES
ROWS_PER_TILE = N_NODES // NUM_SUBCORES

W = 64
DEGW = 8
CHUNK = 400
N_CHUNKS = E_PER_TILE // CHUNK


def _make_sc_aggregate(with_deg: bool):
  mesh = plsc.VectorSubcoreMesh(core_axis_name="c", subcore_axis_name="s",
                                num_cores=NUM_CORES, num_subcores=NUM_SUBCORES)

  out_type = [jax.ShapeDtypeStruct((N_NODES, D), jnp.float32)]
  scratch = [
      pltpu.VMEM((CHUNK,), jnp.int32),
      pltpu.VMEM((CHUNK,), jnp.int32),
      pltpu.VMEM((CHUNK,), jnp.int32),
      pltpu.VMEM((CHUNK,), jnp.int32),
      pltpu.VMEM((CHUNK, W), jnp.float32),
      pltpu.VMEM((CHUNK, W), jnp.float32),
      pltpu.VMEM_SHARED((N_NODES, W), jnp.float32),
      pltpu.SemaphoreType.DMA,
      pltpu.SemaphoreType.DMA,
      pltpu.SemaphoreType.DMA,
      pltpu.SemaphoreType.DMA,
      pltpu.SemaphoreType.DMA,
  ]
  if with_deg:
    out_type.append(jax.ShapeDtypeStruct((N_NODES, D), jnp.float32))
    scratch += [
        pltpu.VMEM((CHUNK, DEGW), jnp.float32),
        pltpu.VMEM_SHARED((N_NODES, DEGW), jnp.float32),
    ]

  @functools.partial(
      pl.kernel,
      out_type=tuple(out_type),
      mesh=mesh,
      scratch_types=scratch,
      compiler_params=pltpu.CompilerParams(use_tc_tiling_on_sc=False),
  )
  def sc_agg(table_hbm, eidx_hbm, zeros_hbm, ones_hbm, *rest):
    if with_deg:
      out_hbm, deg_hbm = rest[0], rest[1]
      (src0, src1, dst0, dst1, buf0, buf1, acc_sh,
       isem0, isem1, rsem0, rsem1, zsem, ones_v, deg_sh) = rest[2:]
    else:
      out_hbm = rest[0]
      (src0, src1, dst0, dst1, buf0, buf1, acc_sh,
       isem0, isem1, rsem0, rsem1, zsem) = rest[1:]

    cid = lax.axis_index("c")
    sid = lax.axis_index("s")
    row0 = sid * ROWS_PER_TILE
    base = sid * E_PER_TILE
    rows = pl.ds(row0, ROWS_PER_TILE)

    ibufs = ((src0, dst0), (src1, dst1))
    isems = (isem0, isem1)
    rbufs = (buf0, buf1)
    rsems = (rsem0, rsem1)

    zcopies = [pltpu.make_async_copy(zeros_hbm.at[rows], acc_sh.at[rows],
                                     zsem)]
    if with_deg:
      zcopies.append(pltpu.make_async_copy(
          zeros_hbm.at[rows, pl.ds(0, DEGW)], deg_sh.at[rows], zsem))
      pltpu.sync_copy(ones_hbm, ones_v)
    for z in zcopies:
      z.start()

    def idx_load(k, b):
      off = base + k * CHUNK
      return (pltpu.make_async_copy(eidx_hbm.at[0, pl.ds(off, CHUNK)],
                                    ibufs[b][0], isems[b]),
              pltpu.make_async_copy(eidx_hbm.at[1, pl.ds(off, CHUNK)],
                                    ibufs[b][1], isems[b]))

    def run_edges(half):
      def transform_src(b):
        sref = ibufs[b][0]

        def tbody(i, carry):
          sv = sref[pl.ds(i * 16, 16)]
          sref[pl.ds(i * 16, 16)] = sv * 2 + cid
          return carry

        lax.fori_loop(0, CHUNK // 16, tbody, 0, unroll=False)

      def gather(b):
        return pltpu.make_async_copy(table_hbm.at[ibufs[b][0]], rbufs[b],
                                     rsems[b])

      ia, ib = idx_load(0, 0)
      ia.start(); ib.start(); ia.wait(); ib.wait()
      transform_src(0)
      ia, ib = idx_load(1, 1)
      ia.start(); ib.start()
      gather(0).start()
      for z in zcopies:
        z.wait()
      plsc.subcore_barrier()

      def body(j, carry):
        for b in range(2):
          k = 2 * j + b

          @pl.when(k + 1 < N_CHUNKS)
          def _():
            ia, ib = idx_load(k + 1, 1 - b)
            ia.wait(); ib.wait()
            transform_src(1 - b)
            gather(1 - b).start()

          gather(b).wait()
          pltpu.sync_copy(rbufs[b], acc_sh.at[ibufs[b][1]], add=True)
          if with_deg and b == half:
            pltpu.sync_copy(ones_v, deg_sh.at[ibufs[b][1]], add=True)

          @pl.when(k + 2 < N_CHUNKS)
          def _():
            ia, ib = idx_load(k + 2, b)
            ia.start(); ib.start()
        return carry

      lax.fori_loop(0, N_CHUNKS // 2, body, 0, unroll=False)

    @pl.when(cid == 0)
    def _():
      run_edges(0)

    @pl.when(cid == 1)
    def _():
      run_edges(1)

    plsc.subcore_barrier()

    @pl.when(cid == 0)
    def _():
      pltpu.sync_copy(acc_sh.at[rows], out_hbm.at[rows, pl.ds(0, W)])
      if with_deg:
        pltpu.sync_copy(deg_sh.at[rows], deg_hbm.at[rows, pl.ds(0, DEGW)])

    @pl.when(cid == 1)
    def _():
      pltpu.sync_copy(acc_sh.at[rows], out_hbm.at[rows, pl.ds(W, W)])
      if with_deg:
        pltpu.sync_copy(deg_sh.at[rows], deg_hbm.at[rows, pl.ds(DEGW, DEGW)])

  return sc_agg


_sc_agg_cache = {}


def _sc_aggregate(with_deg: bool):
  if with_deg not in _sc_agg_cache:
    _sc_agg_cache[with_deg] = _make_sc_aggregate(with_deg)
  return _sc_agg_cache[with_deg]


ROW_BLK = 5000


def _tc_layer1_body(agg_ref, deg_ref, x_ref, ws_ref, wn_ref, b_ref,
                    h1_ref, deginv_ref):
  deg = deg_ref[:, 0:1] + deg_ref[:, DEGW:DEGW + 1]
  scale = 1.0 / jnp.maximum(deg, 1.0)
  hn = agg_ref[...] * scale
  acc = jnp.dot(x_ref[...], ws_ref[...], preferred_element_type=jnp.float32)
  acc += jnp.dot(hn, wn_ref[...], preferred_element_type=jnp.float32)
  acc += b_ref[...]
  h1_ref[...] = jnp.maximum(acc, 0.0)
  deginv_ref[...] = scale


def _tc_layer2_body(agg_ref, h1_ref, deginv_ref, ws_ref, wn_ref,
                    b_ref, out_ref):
  hn = agg_ref[...] * deginv_ref[...]
  acc = jnp.dot(h1_ref[...], ws_ref[...], preferred_element_type=jnp.float32)
  acc += jnp.dot(hn, wn_ref[...], preferred_element_type=jnp.float32)
  acc += b_ref[...]
  out_ref[...] = acc


def _row_spec(width):
  return pl.BlockSpec((ROW_BLK, width), lambda i: (i, 0))


def _full_spec(shape):
  return pl.BlockSpec(shape, lambda i: (0,) * len(shape))


def _tc_layer1(agg, degblk, x, ws, wn, b):
  return pl.pallas_call(
      _tc_layer1_body,
      grid=(N_NODES // ROW_BLK,),
      in_specs=[
          _row_spec(D), _row_spec(D), _row_spec(D),
          _full_spec((D, D)), _full_spec((D, D)), _full_spec((1, D)),
      ],
      out_specs=[_row_spec(D), _row_spec(1)],
      out_shape=[
          jax.ShapeDtypeStruct((N_NODES, D), jnp.float32),
          jax.ShapeDtypeStruct((N_NODES, 1), jnp.float32),
      ],
  )(agg, degblk, x, ws, wn, b)


def _tc_layer2(agg, h1, deginv, ws, wn, b):
  return pl.pallas_call(
      _tc_layer2_body,
      grid=(N_NODES // ROW_BLK,),
      in_specs=[
          _row_spec(D), _row_spec(D), _row_spec(1),
          _full_spec((D, D)), _full_spec((D, D)), _full_spec((1, D)),
      ],
      out_specs=_row_spec(D),
      out_shape=jax.ShapeDtypeStruct((N_NODES, D), jnp.float32),
  )(agg, h1, deginv, ws, wn, b)


@jax.jit
def kernel(inputs, edge_index, W_self1, W_neigh1, b1, W_self2, W_neigh2, b2):
  x2 = inputs.reshape(2 * N_NODES, W)

  zeros = jnp.zeros((N_NODES, W), jnp.float32)
  ones_rows = jnp.ones((CHUNK, DEGW), jnp.float32)

  agg1, degblk = _sc_aggregate(True)(x2, edge_index, zeros, ones_rows)
  h1, deginv = _tc_layer1(agg1, degblk, inputs,
                          W_self1, W_neigh1, b1.reshape(1, D))

  agg2, = _sc_aggregate(False)(h1.reshape(2 * N_NODES, W), edge_index,
                               zeros, ones_rows)
  out = _tc_layer2(agg2, h1, deginv,
                   W_self2, W_neigh2, b2.reshape(1, D))
  return out
